# baseline (speedup 1.0000x reference)
import functools

import jax
import jax.numpy as jnp
from jax import lax
from jax.experimental import pallas as pl
from jax.experimental.pallas import tpu as pltpu
from jax.experimental.pallas import tpu_sc as plsc

T = 2048
H = 768
FF = 3072
E = 8
C = T
TM = 128
M = C // TM
H2 = H // 2
CH = 128
CHS = 64
NC = 2
NS = 16


def _router_body(x_ref, sim_ref, gates_ref, logits_ref, mask_ref, rw_ref,
                 bits_ref):
    x = x_ref[...]
    sim = sim_ref[...]
    g = gates_ref[...]

    ssq = jnp.sum(sim * sim, axis=0, keepdims=True)
    simn = sim / jnp.maximum(jnp.sqrt(ssq), 1e-12)
    xsq = jnp.sum(x * x, axis=1, keepdims=True)
    xn = x / jnp.maximum(jnp.sqrt(xsq), 1e-12)
    logits = jax.lax.dot_general(
        xn.astype(jnp.bfloat16), simn.astype(jnp.bfloat16),
        (((1,), (0,)), ((), ())),
        preferred_element_type=jnp.float32)

    thr = jax.nn.sigmoid(g)
    gated = jnp.maximum(logits - thr, 0.0)
    maskv = (gated > 0.0).astype(jnp.float32)
    num_active = jnp.sum(maskv, axis=1, keepdims=True)

    mx = jnp.max(logits, axis=1, keepdims=True)
    ii = lax.broadcasted_iota(jnp.int32, (T, E), 1)
    cand = jnp.where(logits == mx, ii, E)
    first = jnp.min(cand, axis=1, keepdims=True)
    onehot = (ii == first).astype(jnp.float32)

    mask2 = jnp.where(num_active == 0.0, onehot, maskv)
    glm = jnp.where(mask2 > 0.0, gated, -1e9)
    m2 = jnp.max(glm, axis=1, keepdims=True)
    ex = jnp.exp(glm - m2)
    rw = ex / jnp.sum(ex, axis=1, keepdims=True)

    logits_ref[...] = logits
    mask_ref[...] = mask2
    rw_ref[...] = rw
    pw = (1 << lax.broadcasted_iota(jnp.int32, (1, E), 1)).astype(jnp.float32)
    bits_ref[...] = jnp.sum(mask2 * pw, axis=1, keepdims=True
                            ).astype(jnp.int32)


def _dyn_gather(v, idx):
    return lax.gather(
        v, idx[:, None],
        lax.GatherDimensionNumbers(offset_dims=(), collapsed_slice_dims=(0,),
                                   start_index_map=(0,)),
        slice_sizes=(1,),
        mode=lax.GatherScatterMode.PROMISE_IN_BOUNDS)


def _lane_shift_sum(cs, s):
    lanes = lax.iota(jnp.int32, 16)
    g = _dyn_gather(cs, jnp.maximum(lanes - s, 0))
    return cs + jnp.where(lanes >= s, g, 0)


def _prefix16(b):
    cs = _lane_shift_sum(b, 1)
    cs = _lane_shift_sum(cs, 2)
    cs = _lane_shift_sum(cs, 4)
    cs = _lane_shift_sum(cs, 8)
    return cs


def _splat_last(cs):
    return _dyn_gather(cs, jnp.full((16,), 15, jnp.int32))


def _dispatch_body(bits_hbm, pos_hbm, counts_hbm, bits_v, pos_v, cnt_v):
    wid = lax.axis_index("s") * NC + lax.axis_index("c")

    @pl.when(wid < E)
    def _():
        e = wid
        pltpu.sync_copy(bits_hbm, bits_v)

        def countstep(i, offv):
            v = bits_v[pl.ds(i * 16, 16)]
            b = jnp.bitwise_and(jnp.right_shift(v, e), 1)
            cs = _prefix16(b)
            return offv + _splat_last(cs)
        countv = lax.fori_loop(0, T // 16, countstep,
                               jnp.zeros((16,), jnp.int32))

        def posstep(i, carry):
            offa, offi = carry
            v = bits_v[pl.ds(i * 16, 16)]
            b = jnp.bitwise_and(jnp.right_shift(v, e), 1)
            ca = _prefix16(b)
            ci = _prefix16(1 - b)
            pa = offa + ca - 1
            pi = offi + ci - 1
            pos_v[pl.ds(i * 16, 16)] = pi + b * (pa - pi)
            return (offa + _splat_last(ca), offi + _splat_last(ci))
        lax.fori_loop(0, T // 16, posstep,
                      (jnp.zeros((16,), jnp.int32), countv))

        cnt_v[...] = countv
        pltpu.sync_copy(cnt_v, counts_hbm.at[e])
        pltpu.sync_copy(pos_v, pos_hbm.at[e])


def _make_dispatch():
    mesh = plsc.VectorSubcoreMesh(core_axis_name="c", subcore_axis_name="s",
                                  num_cores=NC, num_subcores=NS)
    return functools.partial(
        pl.kernel, _dispatch_body, mesh=mesh,
        out_type=[
            jax.ShapeDtypeStruct((E, C), jnp.int32),
            jax.ShapeDtypeStruct((E, 16), jnp.int32),
        ],
        scratch_types=[
            pltpu.VMEM((T,), jnp.int32),
            pltpu.VMEM((C,), jnp.int32),
            pltpu.VMEM((16,), jnp.int32),
        ],
    )()


def _permute_body(x_hbm, pos_hbm, xg_hbm, pos_v, didx_v, rows_v, sem, sem2):
    wid = lax.axis_index("s") * NC + lax.axis_index("c")
    e = wid // 4
    base = (wid % 4) * (C // 4)
    pltpu.sync_copy(pos_hbm.at[e, pl.ds(base, C // 4)], pos_v)

    nch = (C // 4) // CH

    def build(j, q):
        for k in range(CH // 16):
            p = pos_v[pl.ds(j * CH + k * 16, 16)]
            didx_v[q, pl.ds(k * 16, 16)] = p + e * C

    def fire_r(j, q):
        return pltpu.async_copy(x_hbm.at[pl.ds(base + j * CH, CH)],
                                rows_v.at[q], sem)

    def fire_w(j, q):
        return pltpu.async_copy(rows_v.at[q], xg_hbm.at[didx_v.at[q]], sem2)

    hr = {0: fire_r(0, 0)}
    hw = {}
    for j in range(nch):
        q = j % 2
        hr[j].wait()
        build(j, q)
        hw[j] = fire_w(j, q)
        if j + 1 < nch:
            if j >= 1:
                hw[j - 1].wait()
            hr[j + 1] = fire_r(j + 1, (j + 1) % 2)
    if nch >= 2:
        hw[nch - 2].wait()
    hw[nch - 1].wait()


def _make_permute():
    mesh = plsc.VectorSubcoreMesh(core_axis_name="c", subcore_axis_name="s",
                                  num_cores=NC, num_subcores=NS)
    return functools.partial(
        pl.kernel, _permute_body, mesh=mesh,
        out_type=jax.ShapeDtypeStruct((E * C, H2), jnp.int32),
        scratch_types=[
            pltpu.VMEM((C // 4,), jnp.int32),
            pltpu.VMEM((2, CH), jnp.int32),
            pltpu.VMEM((2, CH, H2), jnp.int32),
            pltpu.SemaphoreType.DMA,
            pltpu.SemaphoreType.DMA,
        ],
    )()


def _expert_body(cnt_ref, xg_ref, w1_ref, w2_ref, yg_ref):
    m = pl.program_id(1)
    cnt = cnt_ref[pl.program_id(0)]
    nt = (cnt + TM - 1) // TM

    @pl.when(m < nt)
    def _():
        x = xg_ref[...]
        w1 = w1_ref[0]
        h = jax.lax.dot_general(x, w1, (((1,), (1,)), ((), ())),
                                preferred_element_type=jnp.float32)
        a = 0.5 * h * (1.0 + lax.erf(h * 0.7071067811865476))
        w2 = w2_ref[0]
        y = jax.lax.dot_general(
            a.astype(jnp.bfloat16), w2, (((1,), (1,)), ((), ())),
            preferred_element_type=jnp.float32)
        srow = m * TM + lax.broadcasted_iota(jnp.int32, (TM, 1), 0)
        yg_ref[...] = jnp.where(srow < cnt, y, 0.0)

    @pl.when(m >= nt)
    def _():
        yg_ref[...] = jnp.zeros_like(yg_ref)


def _clamped(e, m, cnt_ref):
    nt = (cnt_ref[e] + TM - 1) // TM
    return e * M + jnp.minimum(m, jnp.maximum(nt - 1, 0)), 0


def _scatter_body(yg_hbm, pos_hbm, feo_hbm, buf_v, gidx_v, didx_v, pos_v,
                  sem, sem2):
    wid = lax.axis_index("s") * NC + lax.axis_index("c")
    e = wid // 4
    base = (wid % 4) * (C // 4)
    pltpu.sync_copy(pos_hbm.at[e], pos_v)

    nch = (C // 4) // CHS

    def build(j, q):
        for k in range(CHS // 16):
            p = pos_v[pl.ds(base + j * CHS + k * 16, 16)]
            gidx_v[q, pl.ds(k * 16, 16)] = p + e * C
            toks = lax.iota(jnp.int32, 16) + (base + j * CHS + k * 16)
            didx_v[q, pl.ds(k * 16, 16)] = toks * E + e

    def fire_g(j, q):
        return pltpu.async_copy(yg_hbm.at[gidx_v.at[q]], buf_v.at[q], sem)

    def fire_s(j, q):
        return pltpu.async_copy(buf_v.at[q], feo_hbm.at[didx_v.at[q]], sem2)

    build(0, 0)
    hg = {0: fire_g(0, 0)}
    hs = {}
    for j in range(nch):
        q = j % 2
        hg[j].wait()
        hs[j] = fire_s(j, q)
        if j + 1 < nch:
            if j >= 1:
                hs[j - 1].wait()
            build(j + 1, (j + 1) % 2)
            hg[j + 1] = fire_g(j + 1, (j + 1) % 2)
    if nch >= 2:
        hs[nch - 2].wait()
    hs[nch - 1].wait()


def _make_scatter():
    mesh = plsc.VectorSubcoreMesh(core_axis_name="c", subcore_axis_name="s",
                                  num_cores=NC, num_subcores=NS)
    return functools.partial(
        pl.kernel, _scatter_body, mesh=mesh,
        out_type=jax.ShapeDtypeStruct((T * E, H), jnp.float32),
        scratch_types=[
            pltpu.VMEM((2, CHS, H), jnp.float32),
            pltpu.VMEM((2, CHS), jnp.int32),
            pltpu.VMEM((2, CHS), jnp.int32),
            pltpu.VMEM((C,), jnp.int32),
            pltpu.SemaphoreType.DMA,
            pltpu.SemaphoreType.DMA,
        ],
    )()


def _combine_body(feo_ref, rw_ref, final_ref):
    fe = feo_ref[...].reshape(TM, E, H)
    rw = rw_ref[...]
    acc = jnp.zeros((TM, H), jnp.float32)
    for e in range(E):
        eo = (lax.broadcasted_iota(jnp.int32, (E, 1), 0) == e
              ).astype(jnp.float32)
        rcol = jnp.dot(rw, eo, preferred_element_type=jnp.float32)
        acc = acc + rcol * fe[:, e, :]
    final_ref[...] = acc


@jax.jit
def kernel(hidden_states, sim_matrix, gates, W1, W2):
    x = hidden_states
    gates2d = gates.reshape(1, E)

    logits, mask, rw, bits2d = pl.pallas_call(
        _router_body,
        out_shape=[
            jax.ShapeDtypeStruct((T, E), jnp.float32),
            jax.ShapeDtypeStruct((T, E), jnp.float32),
            jax.ShapeDtypeStruct((T, E), jnp.float32),
            jax.ShapeDtypeStruct((T, 1), jnp.int32),
        ],
    )(x, sim_matrix, gates2d)

    xb = x.astype(jnp.bfloat16)
    xi = lax.bitcast_convert_type(xb.reshape(T, H2, 2), jnp.int32)
    bits = bits2d.reshape(T)

    pos, counts16 = _make_dispatch()(bits)
    xg_i32 = _make_permute()(xi, pos)
    xg = lax.bitcast_convert_type(xg_i32, jnp.bfloat16).reshape(E * C, H)
    counts = counts16[:, 0]

    W1b = W1.astype(jnp.bfloat16)
    W2b = W2.astype(jnp.bfloat16)

    yg = pl.pallas_call(
        _expert_body,
        grid_spec=pltpu.PrefetchScalarGridSpec(
            num_scalar_prefetch=1,
            grid=(E, M),
            in_specs=[
                pl.BlockSpec((TM, H), _clamped),
                pl.BlockSpec((1, FF, H), lambda e, m, c: (e, 0, 0)),
                pl.BlockSpec((1, H, FF), lambda e, m, c: (e, 0, 0)),
            ],
            out_specs=pl.BlockSpec((TM, H), lambda e, m, c: (e * M + m, 0)),
        ),
        out_shape=jax.ShapeDtypeStruct((E * C + 8, H), jnp.float32),
        compiler_params=pltpu.CompilerParams(
            dimension_semantics=("arbitrary", "arbitrary"),
            vmem_limit_bytes=100 * 1024 * 1024,
        ),
    )(counts, xg, W1b, W2b)

    feo_flat = _make_scatter()(yg, pos)

    final = pl.pallas_call(
        _combine_body,
        grid=(T // TM,),
        in_specs=[
            pl.BlockSpec((TM * E, H), lambda t: (t, 0)),
            pl.BlockSpec((TM, E), lambda t: (t, 0)),
        ],
        out_specs=pl.BlockSpec((TM, H), lambda t: (t, 0)),
        out_shape=jax.ShapeDtypeStruct((T, H), jnp.float32),
    )(feo_flat, rw)

    feo = feo_flat.reshape(T, E, H)
    return (final, feo, logits, mask)


# speedup vs baseline: 1.1034x; 1.1034x over previous
import functools

import jax
import jax.numpy as jnp
from jax import lax
from jax.experimental import pallas as pl
from jax.experimental.pallas import tpu as pltpu
from jax.experimental.pallas import tpu_sc as plsc

T = 2048
H = 768
FF = 3072
E = 8
C = T
TM = 256
M = C // TM
H2 = H // 2
CH = 128
CHS = 64
NC = 2
NS = 16


def _router_body(x_ref, sim_ref, gates_ref, logits_ref, mask_ref, rw_ref,
                 bits_ref):
    x = x_ref[...]
    sim = sim_ref[...]
    g = gates_ref[...]

    ssq = jnp.sum(sim * sim, axis=0, keepdims=True)
    simn = sim / jnp.maximum(jnp.sqrt(ssq), 1e-12)
    xsq = jnp.sum(x * x, axis=1, keepdims=True)
    xn = x / jnp.maximum(jnp.sqrt(xsq), 1e-12)
    logits = jax.lax.dot_general(
        xn.astype(jnp.bfloat16), simn.astype(jnp.bfloat16),
        (((1,), (0,)), ((), ())),
        preferred_element_type=jnp.float32)

    thr = jax.nn.sigmoid(g)
    gated = jnp.maximum(logits - thr, 0.0)
    maskv = (gated > 0.0).astype(jnp.float32)
    num_active = jnp.sum(maskv, axis=1, keepdims=True)

    mx = jnp.max(logits, axis=1, keepdims=True)
    ii = lax.broadcasted_iota(jnp.int32, (T, E), 1)
    cand = jnp.where(logits == mx, ii, E)
    first = jnp.min(cand, axis=1, keepdims=True)
    onehot = (ii == first).astype(jnp.float32)

    mask2 = jnp.where(num_active == 0.0, onehot, maskv)
    glm = jnp.where(mask2 > 0.0, gated, -1e9)
    m2 = jnp.max(glm, axis=1, keepdims=True)
    ex = jnp.exp(glm - m2)
    rw = ex / jnp.sum(ex, axis=1, keepdims=True)

    logits_ref[...] = logits
    mask_ref[...] = mask2
    rw_ref[...] = rw
    pw = (1 << lax.broadcasted_iota(jnp.int32, (1, E), 1)).astype(jnp.float32)
    bits_ref[...] = jnp.sum(mask2 * pw, axis=1, keepdims=True
                            ).astype(jnp.int32)


def _dyn_gather(v, idx):
    return lax.gather(
        v, idx[:, None],
        lax.GatherDimensionNumbers(offset_dims=(), collapsed_slice_dims=(0,),
                                   start_index_map=(0,)),
        slice_sizes=(1,),
        mode=lax.GatherScatterMode.PROMISE_IN_BOUNDS)


def _lane_shift_sum(cs, s):
    lanes = lax.iota(jnp.int32, 16)
    g = _dyn_gather(cs, jnp.maximum(lanes - s, 0))
    return cs + jnp.where(lanes >= s, g, 0)


def _prefix16(b):
    cs = _lane_shift_sum(b, 1)
    cs = _lane_shift_sum(cs, 2)
    cs = _lane_shift_sum(cs, 4)
    cs = _lane_shift_sum(cs, 8)
    return cs


def _splat_last(cs):
    return _dyn_gather(cs, jnp.full((16,), 15, jnp.int32))


def _dispatch_body(bits_hbm, pos_hbm, counts_hbm, bits_v, pos_v, cnt_v):
    wid = lax.axis_index("s") * NC + lax.axis_index("c")

    @pl.when(wid < E)
    def _():
        e = wid
        pltpu.sync_copy(bits_hbm, bits_v)

        def countstep(i, offv):
            v = bits_v[pl.ds(i * 16, 16)]
            b = jnp.bitwise_and(jnp.right_shift(v, e), 1)
            cs = _prefix16(b)
            return offv + _splat_last(cs)
        countv = lax.fori_loop(0, T // 16, countstep,
                               jnp.zeros((16,), jnp.int32))

        def posstep(i, carry):
            offa, offi = carry
            v = bits_v[pl.ds(i * 16, 16)]
            b = jnp.bitwise_and(jnp.right_shift(v, e), 1)
            ca = _prefix16(b)
            ci = _prefix16(1 - b)
            pa = offa + ca - 1
            pi = offi + ci - 1
            pos_v[pl.ds(i * 16, 16)] = pi + b * (pa - pi)
            return (offa + _splat_last(ca), offi + _splat_last(ci))
        lax.fori_loop(0, T // 16, posstep,
                      (jnp.zeros((16,), jnp.int32), countv))

        cnt_v[...] = countv
        pltpu.sync_copy(cnt_v, counts_hbm.at[e])
        pltpu.sync_copy(pos_v, pos_hbm.at[e])


def _make_dispatch():
    mesh = plsc.VectorSubcoreMesh(core_axis_name="c", subcore_axis_name="s",
                                  num_cores=NC, num_subcores=NS)
    return functools.partial(
        pl.kernel, _dispatch_body, mesh=mesh,
        out_type=[
            jax.ShapeDtypeStruct((E, C), jnp.int32),
            jax.ShapeDtypeStruct((E, 16), jnp.int32),
        ],
        scratch_types=[
            pltpu.VMEM((T,), jnp.int32),
            pltpu.VMEM((C,), jnp.int32),
            pltpu.VMEM((16,), jnp.int32),
        ],
    )()


def _permute_body(x_hbm, pos_hbm, xg_hbm, pos_v, didx_v, rows_v, sem, sem2):
    wid = lax.axis_index("s") * NC + lax.axis_index("c")
    e = wid // 4
    base = (wid % 4) * (C // 4)
    pltpu.sync_copy(pos_hbm.at[e, pl.ds(base, C // 4)], pos_v)

    nch = (C // 4) // CH

    def build(j, q):
        for k in range(CH // 16):
            p = pos_v[pl.ds(j * CH + k * 16, 16)]
            didx_v[q, pl.ds(k * 16, 16)] = p + e * C

    def fire_r(j, q):
        return pltpu.async_copy(x_hbm.at[pl.ds(base + j * CH, CH)],
                                rows_v.at[q], sem)

    def fire_w(j, q):
        return pltpu.async_copy(rows_v.at[q], xg_hbm.at[didx_v.at[q]], sem2)

    hr = {0: fire_r(0, 0)}
    hw = {}
    for j in range(nch):
        q = j % 2
        hr[j].wait()
        build(j, q)
        hw[j] = fire_w(j, q)
        if j + 1 < nch:
            if j >= 1:
                hw[j - 1].wait()
            hr[j + 1] = fire_r(j + 1, (j + 1) % 2)
    if nch >= 2:
        hw[nch - 2].wait()
    hw[nch - 1].wait()


def _make_permute():
    mesh = plsc.VectorSubcoreMesh(core_axis_name="c", subcore_axis_name="s",
                                  num_cores=NC, num_subcores=NS)
    return functools.partial(
        pl.kernel, _permute_body, mesh=mesh,
        out_type=jax.ShapeDtypeStruct((E * C, H2), jnp.int32),
        scratch_types=[
            pltpu.VMEM((C // 4,), jnp.int32),
            pltpu.VMEM((2, CH), jnp.int32),
            pltpu.VMEM((2, CH, H2), jnp.int32),
            pltpu.SemaphoreType.DMA,
            pltpu.SemaphoreType.DMA,
        ],
    )()


def _expert_body(cnt_ref, xg_ref, w1_ref, w2_ref, yg_ref):
    m = pl.program_id(1)
    cnt = cnt_ref[pl.program_id(0)]
    nt = (cnt + TM - 1) // TM

    @pl.when(m < nt)
    def _():
        x = xg_ref[...]
        w1 = w1_ref[0]
        h = jax.lax.dot_general(x, w1, (((1,), (1,)), ((), ())),
                                preferred_element_type=jnp.float32)
        a = 0.5 * h * (1.0 + lax.erf(h * 0.7071067811865476))
        w2 = w2_ref[0]
        y = jax.lax.dot_general(
            a.astype(jnp.bfloat16), w2, (((1,), (1,)), ((), ())),
            preferred_element_type=jnp.float32)
        srow = m * TM + lax.broadcasted_iota(jnp.int32, (TM, 1), 0)
        yg_ref[...] = jnp.where(srow < cnt, y, 0.0)

    @pl.when(m >= nt)
    def _():
        yg_ref[...] = jnp.zeros_like(yg_ref)


def _clamped(e, m, cnt_ref):
    nt = (cnt_ref[e] + TM - 1) // TM
    return e * M + jnp.minimum(m, jnp.maximum(nt - 1, 0)), 0


def _scatter_body(yg_hbm, pos_hbm, feo_hbm, buf_v, gidx_v, didx_v, pos_v,
                  sem, sem2):
    wid = lax.axis_index("s") * NC + lax.axis_index("c")
    e = wid // 4
    base = (wid % 4) * (C // 4)
    pltpu.sync_copy(pos_hbm.at[e], pos_v)

    nch = (C // 4) // CHS

    def build(j, q):
        for k in range(CHS // 16):
            p = pos_v[pl.ds(base + j * CHS + k * 16, 16)]
            gidx_v[q, pl.ds(k * 16, 16)] = p + e * C
            toks = lax.iota(jnp.int32, 16) + (base + j * CHS + k * 16)
            didx_v[q, pl.ds(k * 16, 16)] = toks * E + e

    def fire_g(j, q):
        return pltpu.async_copy(yg_hbm.at[gidx_v.at[q]], buf_v.at[q], sem)

    def fire_s(j, q):
        return pltpu.async_copy(buf_v.at[q], feo_hbm.at[didx_v.at[q]], sem2)

    build(0, 0)
    hg = {0: fire_g(0, 0)}
    hs = {}
    for j in range(nch):
        q = j % 2
        hg[j].wait()
        hs[j] = fire_s(j, q)
        if j + 1 < nch:
            if j >= 1:
                hs[j - 1].wait()
            build(j + 1, (j + 1) % 2)
            hg[j + 1] = fire_g(j + 1, (j + 1) % 2)
    if nch >= 2:
        hs[nch - 2].wait()
    hs[nch - 1].wait()


def _make_scatter():
    mesh = plsc.VectorSubcoreMesh(core_axis_name="c", subcore_axis_name="s",
                                  num_cores=NC, num_subcores=NS)
    return functools.partial(
        pl.kernel, _scatter_body, mesh=mesh,
        out_type=jax.ShapeDtypeStruct((T * E, H), jnp.float32),
        scratch_types=[
            pltpu.VMEM((2, CHS, H), jnp.float32),
            pltpu.VMEM((2, CHS), jnp.int32),
            pltpu.VMEM((2, CHS), jnp.int32),
            pltpu.VMEM((C,), jnp.int32),
            pltpu.SemaphoreType.DMA,
            pltpu.SemaphoreType.DMA,
        ],
    )()


def _combine_body(feo_ref, rw_ref, final_ref):
    fe = feo_ref[...].reshape(TM, E, H)
    rw = rw_ref[...]
    acc = jnp.zeros((TM, H), jnp.float32)
    for e in range(E):
        eo = (lax.broadcasted_iota(jnp.int32, (E, 1), 0) == e
              ).astype(jnp.float32)
        rcol = jnp.dot(rw, eo, preferred_element_type=jnp.float32)
        acc = acc + rcol * fe[:, e, :]
    final_ref[...] = acc


@jax.jit
def kernel(hidden_states, sim_matrix, gates, W1, W2):
    x = hidden_states
    gates2d = gates.reshape(1, E)

    logits, mask, rw, bits2d = pl.pallas_call(
        _router_body,
        out_shape=[
            jax.ShapeDtypeStruct((T, E), jnp.float32),
            jax.ShapeDtypeStruct((T, E), jnp.float32),
            jax.ShapeDtypeStruct((T, E), jnp.float32),
            jax.ShapeDtypeStruct((T, 1), jnp.int32),
        ],
    )(x, sim_matrix, gates2d)

    xb = x.astype(jnp.bfloat16)
    xi = lax.bitcast_convert_type(xb.reshape(T, H2, 2), jnp.int32)
    bits = bits2d.reshape(T)

    pos, counts16 = _make_dispatch()(bits)
    xg_i32 = _make_permute()(xi, pos)
    xg = lax.bitcast_convert_type(xg_i32, jnp.bfloat16).reshape(E * C, H)
    counts = counts16[:, 0]

    W1b = W1.astype(jnp.bfloat16)
    W2b = W2.astype(jnp.bfloat16)

    yg = pl.pallas_call(
        _expert_body,
        grid_spec=pltpu.PrefetchScalarGridSpec(
            num_scalar_prefetch=1,
            grid=(E, M),
            in_specs=[
                pl.BlockSpec((TM, H), _clamped),
                pl.BlockSpec((1, FF, H), lambda e, m, c: (e, 0, 0)),
                pl.BlockSpec((1, H, FF), lambda e, m, c: (e, 0, 0)),
            ],
            out_specs=pl.BlockSpec((TM, H), lambda e, m, c: (e * M + m, 0)),
        ),
        out_shape=jax.ShapeDtypeStruct((E * C + 8, H), jnp.float32),
        compiler_params=pltpu.CompilerParams(
            dimension_semantics=("arbitrary", "arbitrary"),
            vmem_limit_bytes=100 * 1024 * 1024,
        ),
    )(counts, xg, W1b, W2b)

    feo_flat = _make_scatter()(yg, pos)

    final = pl.pallas_call(
        _combine_body,
        grid=(T // TM,),
        in_specs=[
            pl.BlockSpec((TM * E, H), lambda t: (t, 0)),
            pl.BlockSpec((TM, E), lambda t: (t, 0)),
        ],
        out_specs=pl.BlockSpec((TM, H), lambda t: (t, 0)),
        out_shape=jax.ShapeDtypeStruct((T, H), jnp.float32),
    )(feo_flat, rw)

    feo = feo_flat.reshape(T, E, H)
    return (final, feo, logits, mask)
